# baseline (speedup 1.0000x reference)
import functools

import jax
import jax.numpy as jnp
from jax import lax
from jax.experimental import pallas as pl
from jax.experimental.pallas import tpu as pltpu
from jax.experimental.pallas import tpu_sc as plsc

NC = 2
NS = 16
NW = NC * NS

CH = 128
NB = 8


def _transpose_table(table_t, tail_lin, v, d):
    mesh = plsc.VectorSubcoreMesh(core_axis_name="c", subcore_axis_name="s")
    ntiles = v // 128
    tail = v - ntiles * 128

    @functools.partial(
        pl.kernel,
        mesh=mesh,
        out_type=jax.ShapeDtypeStruct((v * d,), jnp.float32),
        scratch_types=[
            pltpu.VMEM((d, 129), jnp.float32),
            pltpu.VMEM((128 * d,), jnp.float32),
        ],
        compiler_params=pltpu.CompilerParams(
            use_tc_tiling_on_sc=True, needs_layout_passes=False),
    )
    def k(tab_hbm, tail_hbm, out_hbm, slab_v, lin_v):
        wid = lax.axis_index("s") * NC + lax.axis_index("c")
        q, r = divmod(ntiles, NW)
        start = wid * q + jnp.minimum(wid, r)
        cnt = q + (wid < r).astype(jnp.int32)

        @pl.loop(start, start + cnt)
        def _(t):
            pltpu.sync_copy(tab_hbm.at[:, pl.ds(t * 128, 128)],
                            slab_v.at[:, pl.ds(0, 128)])

            @pl.loop(0, 128, unroll=4)
            def _(c):
                for dg in range(d // 16):
                    vals = plsc.load_gather(
                        slab_v,
                        [lax.iota(jnp.int32, 16) + 16 * dg,
                         jnp.full((16,), c, jnp.int32)])
                    lin_v[pl.ds(c * d + 16 * dg, 16)] = vals

            pltpu.sync_copy(lin_v, out_hbm.at[pl.ds(t * 128 * d, 128 * d)])

        if tail:
            @pl.when(wid == NW - 1)
            def _():
                pltpu.sync_copy(tail_hbm, lin_v.at[pl.ds(0, tail * d)])
                pltpu.sync_copy(
                    lin_v.at[pl.ds(0, tail * d)],
                    out_hbm.at[pl.ds(ntiles * 128 * d, tail * d)])

    return k(table_t, tail_lin)


@functools.partial(jax.jit, static_argnums=(2, 3, 4))
def _emb_lookup(xr, table, total, d, nch):
    mesh = plsc.VectorSubcoreMesh(core_axis_name="c", subcore_axis_name="s")
    b_per_w = nch * CH

    @functools.partial(
        pl.kernel,
        mesh=mesh,
        out_type=jax.ShapeDtypeStruct((total, d), jnp.float32),
        scratch_types=[
            pltpu.VMEM((nch, CH), jnp.int32),
            pltpu.VMEM((NB, CH, d), jnp.float32),
            pltpu.SemaphoreType.DMA,
            pltpu.SemaphoreType.DMA,
        ],
        compiler_params=pltpu.CompilerParams(use_tc_tiling_on_sc=False),
    )
    def k(x_hbm, tab_hbm, out_hbm, idx_v, rows_v, gsem, ssem):
        wid = lax.axis_index("s") * NC + lax.axis_index("c")
        base = wid * b_per_w
        pltpu.sync_copy(x_hbm.at[wid], idx_v)

        def gather(c, b):
            pltpu.async_copy(tab_hbm.at[idx_v.at[c]], rows_v.at[b], gsem)

        def wait_gather(b):
            pltpu.make_async_copy(
                tab_hbm.at[idx_v.at[0]], rows_v.at[b], gsem).wait()

        def wait_scatter():
            pltpu.make_async_copy(
                rows_v.at[0], out_hbm.at[pl.ds(base, CH)], ssem).wait()

        for b in range(NB):
            gather(b, b)

        @pl.loop(0, nch // NB)
        def _(p):
            for b in range(NB):
                s = p * NB + b
                wait_gather(b)
                pltpu.async_copy(
                    rows_v.at[b], out_hbm.at[pl.ds(base + s * CH, CH)], ssem)
                @pl.when(jnp.logical_and(s >= 2, s < nch - NB + 2))
                def _():
                    wait_scatter()
                    gather(s + NB - 2, (b - 2) % NB)

        for _ in range(NB):
            wait_scatter()

    v = table.shape[0]
    tail_lin = table[(v // 128) * 128:, :].reshape(-1)
    table_lin = _transpose_table(table.T, tail_lin, v, d).reshape(v, d)
    return k(xr, table_lin)


def kernel(x, embeddings):
    b, h = x.shape
    _, d = embeddings.shape
    total = b * h
    b_per_w = total // NW
    nch = b_per_w // CH
    xr = x.reshape(NW, nch, CH).astype(jnp.int32)
    out = _emb_lookup(xr, embeddings, total, d, nch)
    return out.reshape(b, h, d)


# speedup vs baseline: 1.1948x; 1.1948x over previous
import functools

import jax
import jax.numpy as jnp
from jax import lax
from jax.experimental import pallas as pl
from jax.experimental.pallas import tpu as pltpu
from jax.experimental.pallas import tpu_sc as plsc

NC = 2
NS = 16
NW = NC * NS

CH = 128
NB = 8


def _transpose_table(table_t, tail_lin, v, d):
    mesh = plsc.VectorSubcoreMesh(core_axis_name="c", subcore_axis_name="s")
    ntiles = v // 128
    tail = v - ntiles * 128

    per_w = -(-ntiles // NW)
    if per_w % 2:
        per_w += 1

    @functools.partial(
        pl.kernel,
        mesh=mesh,
        out_type=jax.ShapeDtypeStruct((v * d,), jnp.float32),
        scratch_types=[
            pltpu.VMEM((d, 129), jnp.float32),
            pltpu.VMEM((d, 129), jnp.float32),
            pltpu.VMEM((128 * d,), jnp.float32),
            pltpu.VMEM((128 * d,), jnp.float32),
            pltpu.SemaphoreType.DMA,
            pltpu.SemaphoreType.DMA,
        ],
        compiler_params=pltpu.CompilerParams(
            use_tc_tiling_on_sc=True, needs_layout_passes=False),
    )
    def k(tab_hbm, tail_hbm, out_hbm, slab0, slab1, lin0, lin1, isem, osem):
        slabs = (slab0, slab1)
        lins = (lin0, lin1)
        wid = lax.axis_index("s") * NC + lax.axis_index("c")
        first = wid * per_w

        def tile_of(i):
            return jnp.minimum(first + i, ntiles - 1)

        def slab_in(i, par):
            pltpu.async_copy(
                tab_hbm.at[:, pl.ds(tile_of(i) * 128, 128)],
                slabs[par].at[:, pl.ds(0, 128)], isem)

        def wait_slab(par):
            pltpu.make_async_copy(
                tab_hbm.at[:, pl.ds(0, 128)],
                slabs[par].at[:, pl.ds(0, 128)], isem).wait()

        def wait_out(par):
            pltpu.make_async_copy(
                lins[par], out_hbm.at[pl.ds(0, 128 * d)], osem).wait()

        slab_in(0, 0)
        slab_in(1, 1)

        @pl.loop(0, per_w // 2)
        def _(p):
            for par in range(2):
                i = p * 2 + par
                wait_slab(par)

                @pl.when(i >= 2)
                def _():
                    wait_out(par)

                @pl.loop(0, 128, unroll=4)
                def _(c):
                    for dg in range(d // 16):
                        vals = plsc.load_gather(
                            slabs[par],
                            [lax.iota(jnp.int32, 16) + 16 * dg,
                             jnp.full((16,), c, jnp.int32)])
                        lins[par][pl.ds(c * d + 16 * dg, 16)] = vals

                pltpu.async_copy(
                    lins[par],
                    out_hbm.at[pl.ds(tile_of(i) * 128 * d, 128 * d)], osem)

                @pl.when(i + 2 < per_w)
                def _():
                    slab_in(i + 2, par)

        wait_out(0)
        wait_out(1)

        if tail:
            @pl.when(wid == NW - 1)
            def _():
                pltpu.sync_copy(tail_hbm, lin0.at[pl.ds(0, tail * d)])
                pltpu.sync_copy(
                    lin0.at[pl.ds(0, tail * d)],
                    out_hbm.at[pl.ds(ntiles * 128 * d, tail * d)])

    return k(table_t, tail_lin)


@functools.partial(jax.jit, static_argnums=(2, 3, 4))
def _emb_lookup(xr, table, total, d, nch):
    mesh = plsc.VectorSubcoreMesh(core_axis_name="c", subcore_axis_name="s")
    b_per_w = nch * CH

    @functools.partial(
        pl.kernel,
        mesh=mesh,
        out_type=jax.ShapeDtypeStruct((total, d), jnp.float32),
        scratch_types=[
            pltpu.VMEM((nch, CH), jnp.int32),
            pltpu.VMEM((NB, CH, d), jnp.float32),
            pltpu.SemaphoreType.DMA,
            pltpu.SemaphoreType.DMA,
        ],
        compiler_params=pltpu.CompilerParams(use_tc_tiling_on_sc=False),
    )
    def k(x_hbm, tab_hbm, out_hbm, idx_v, rows_v, gsem, ssem):
        wid = lax.axis_index("s") * NC + lax.axis_index("c")
        base = wid * b_per_w
        pltpu.sync_copy(x_hbm.at[wid], idx_v)

        def gather(c, b):
            pltpu.async_copy(tab_hbm.at[idx_v.at[c]], rows_v.at[b], gsem)

        def wait_gather(b):
            pltpu.make_async_copy(
                tab_hbm.at[idx_v.at[0]], rows_v.at[b], gsem).wait()

        def wait_scatter():
            pltpu.make_async_copy(
                rows_v.at[0], out_hbm.at[pl.ds(base, CH)], ssem).wait()

        for b in range(NB):
            gather(b, b)

        @pl.loop(0, nch // NB)
        def _(p):
            for b in range(NB):
                s = p * NB + b
                wait_gather(b)
                pltpu.async_copy(
                    rows_v.at[b], out_hbm.at[pl.ds(base + s * CH, CH)], ssem)
                @pl.when(jnp.logical_and(s >= 2, s < nch - NB + 2))
                def _():
                    wait_scatter()
                    gather(s + NB - 2, (b - 2) % NB)

        for _ in range(NB):
            wait_scatter()

    v = table.shape[0]
    tail_lin = table[(v // 128) * 128:, :].reshape(-1)
    table_lin = _transpose_table(table.T, tail_lin, v, d).reshape(v, d)
    return k(xr, table_lin)


def kernel(x, embeddings):
    b, h = x.shape
    _, d = embeddings.shape
    total = b * h
    b_per_w = total // NW
    nch = b_per_w // CH
    xr = x.reshape(NW, nch, CH).astype(jnp.int32)
    out = _emb_lookup(xr, embeddings, total, d, nch)
    return out.reshape(b, h, d)


# speedup vs baseline: 1.2107x; 1.0133x over previous
import functools

import jax
import jax.numpy as jnp
from jax import lax
from jax.experimental import pallas as pl
from jax.experimental.pallas import tpu as pltpu
from jax.experimental.pallas import tpu_sc as plsc

NC = 2
NS = 16
NW = NC * NS

CH = 128
NB = 8


def _transpose_table_tc(table_t, v, d):
    w = 1024
    h = v // 2
    grid = (h + w - 1) // w
    t3 = table_t.reshape(d, 2, h)

    def body(in_ref, out_ref):
        out_ref[...] = jnp.concatenate(
            [in_ref[:, 0, :].T, in_ref[:, 1, :].T], axis=1)

    return pl.pallas_call(
        body,
        grid=(grid,),
        in_specs=[pl.BlockSpec((d, 2, w), lambda i: (0, 0, i))],
        out_specs=pl.BlockSpec((w, 2 * d), lambda i: (i, 0)),
        out_shape=jax.ShapeDtypeStruct((h, 2 * d), jnp.float32),
    )(t3)


@functools.partial(jax.jit, static_argnums=(2, 3, 4))
def _emb_lookup(xr, table, total, d, nch):
    mesh = plsc.VectorSubcoreMesh(core_axis_name="c", subcore_axis_name="s")
    b_per_w = nch * CH
    half = table.shape[0] // 2

    @functools.partial(
        pl.kernel,
        mesh=mesh,
        out_type=jax.ShapeDtypeStruct((total, d), jnp.float32),
        scratch_types=[
            pltpu.VMEM((nch, CH), jnp.int32),
            pltpu.VMEM((NB, CH, d), jnp.float32),
            pltpu.SemaphoreType.DMA,
            pltpu.SemaphoreType.DMA,
        ],
        compiler_params=pltpu.CompilerParams(use_tc_tiling_on_sc=False),
    )
    def k(x_hbm, tab_hbm, out_hbm, idx_v, rows_v, gsem, ssem):
        wid = lax.axis_index("s") * NC + lax.axis_index("c")
        base = wid * b_per_w
        pltpu.sync_copy(x_hbm.at[wid], idx_v)

        @pl.loop(0, nch)
        def _(c):
            row = idx_v.at[c]
            for g in range(CH // 16):
                vv = row[pl.ds(g * 16, 16)]
                hi = jnp.full((16,), 2 * half - 1, jnp.int32)
                lo = jnp.full((16,), 0, jnp.int32)
                off = jnp.where(vv >= jnp.full((16,), half, jnp.int32), hi, lo)
                row[pl.ds(g * 16, 16)] = vv + vv - off

        def gather(c, b):
            pltpu.async_copy(tab_hbm.at[idx_v.at[c]], rows_v.at[b], gsem)

        def wait_gather(b):
            pltpu.make_async_copy(
                tab_hbm.at[idx_v.at[0]], rows_v.at[b], gsem).wait()

        def wait_scatter():
            pltpu.make_async_copy(
                rows_v.at[0], out_hbm.at[pl.ds(base, CH)], ssem).wait()

        for b in range(NB):
            gather(b, b)

        @pl.loop(0, nch // NB)
        def _(p):
            for b in range(NB):
                s = p * NB + b
                wait_gather(b)
                pltpu.async_copy(
                    rows_v.at[b], out_hbm.at[pl.ds(base + s * CH, CH)], ssem)
                @pl.when(jnp.logical_and(s >= 2, s < nch - NB + 2))
                def _():
                    wait_scatter()
                    gather(s + NB - 2, (b - 2) % NB)

        for _ in range(NB):
            wait_scatter()

    v = table.shape[0]
    table_lin = _transpose_table_tc(table.T, v, d).reshape(v, d)
    return k(xr, table_lin)


def kernel(x, embeddings):
    b, h = x.shape
    _, d = embeddings.shape
    total = b * h
    b_per_w = total // NW
    nch = b_per_w // CH
    xr = x.reshape(NW, nch, CH).astype(jnp.int32)
    out = _emb_lookup(xr, embeddings, total, d, nch)
    return out.reshape(b, h, d)


# speedup vs baseline: 1.4085x; 1.1634x over previous
import functools

import jax
import jax.numpy as jnp
from jax import lax
from jax.experimental import pallas as pl
from jax.experimental.pallas import tpu as pltpu
from jax.experimental.pallas import tpu_sc as plsc

NC = 2
NS = 16
NW = NC * NS

CH = 128
NB = 8


def _transpose_table_tc(table_t, v, d):
    w = 8192
    h = v // 2
    grid = (h + w - 1) // w
    t3 = table_t.reshape(d, 2, h)

    def body(in_ref, out_ref):
        out_ref[...] = jnp.concatenate(
            [in_ref[:, 0, :].T, in_ref[:, 1, :].T], axis=1)

    return pl.pallas_call(
        body,
        grid=(grid,),
        in_specs=[pl.BlockSpec((d, 2, w), lambda i: (0, 0, i))],
        out_specs=pl.BlockSpec((w, 2 * d), lambda i: (i, 0)),
        out_shape=jax.ShapeDtypeStruct((h, 2 * d), jnp.float32),
    )(t3)


@functools.partial(jax.jit, static_argnums=(2, 3, 4))
def _emb_lookup(xr, table, total, d, nch):
    mesh = plsc.VectorSubcoreMesh(core_axis_name="c", subcore_axis_name="s")
    b_per_w = nch * CH
    half = table.shape[0] // 2

    @functools.partial(
        pl.kernel,
        mesh=mesh,
        out_type=jax.ShapeDtypeStruct((total, d), jnp.float32),
        scratch_types=[
            pltpu.VMEM((nch, CH), jnp.int32),
            pltpu.VMEM((NB, CH, d), jnp.float32),
            pltpu.SemaphoreType.DMA,
            pltpu.SemaphoreType.DMA,
        ],
        compiler_params=pltpu.CompilerParams(use_tc_tiling_on_sc=False),
    )
    def k(x_hbm, tab_hbm, out_hbm, idx_v, rows_v, gsem, ssem):
        wid = lax.axis_index("s") * NC + lax.axis_index("c")
        base = wid * b_per_w
        pltpu.sync_copy(x_hbm.at[wid], idx_v)

        @pl.loop(0, nch)
        def _(c):
            row = idx_v.at[c]
            for g in range(CH // 16):
                vv = row[pl.ds(g * 16, 16)]
                hi = jnp.full((16,), 2 * half - 1, jnp.int32)
                lo = jnp.full((16,), 0, jnp.int32)
                off = jnp.where(vv >= jnp.full((16,), half, jnp.int32), hi, lo)
                row[pl.ds(g * 16, 16)] = vv + vv - off

        def gather(c, b):
            pltpu.async_copy(tab_hbm.at[idx_v.at[c]], rows_v.at[b], gsem)

        def wait_gather(b):
            pltpu.make_async_copy(
                tab_hbm.at[idx_v.at[0]], rows_v.at[b], gsem).wait()

        def wait_scatter():
            pltpu.make_async_copy(
                rows_v.at[0], out_hbm.at[pl.ds(base, CH)], ssem).wait()

        for b in range(NB):
            gather(b, b)

        @pl.loop(0, nch // NB)
        def _(p):
            for b in range(NB):
                s = p * NB + b
                wait_gather(b)
                pltpu.async_copy(
                    rows_v.at[b], out_hbm.at[pl.ds(base + s * CH, CH)], ssem)
                @pl.when(jnp.logical_and(s >= 2, s < nch - NB + 2))
                def _():
                    wait_scatter()
                    gather(s + NB - 2, (b - 2) % NB)

        for _ in range(NB):
            wait_scatter()

    v = table.shape[0]
    table_lin = _transpose_table_tc(table.T, v, d).reshape(v, d)
    return k(xr, table_lin)


def kernel(x, embeddings):
    b, h = x.shape
    _, d = embeddings.shape
    total = b * h
    b_per_w = total // NW
    nch = b_per_w // CH
    xr = x.reshape(NW, nch, CH).astype(jnp.int32)
    out = _emb_lookup(xr, embeddings, total, d, nch)
    return out.reshape(b, h, d)


# speedup vs baseline: 4.0666x; 2.8872x over previous
import functools

import jax
import jax.numpy as jnp
from jax import lax
from jax.experimental import pallas as pl
from jax.experimental.pallas import tpu as pltpu
from jax.experimental.pallas import tpu_sc as plsc

NC = 2
NS = 16
NW = NC * NS

CH = 128
NB = 8


PAIR_W = 8192


def _transpose_table_tc(table_t, v, d):
    w = PAIR_W
    grid = (v + 2 * w - 1) // (2 * w)
    h = grid * w
    last = (v + w - 1) // w - 1

    def body(in0_ref, in1_ref, out_ref):
        out_ref[...] = jnp.concatenate(
            [in0_ref[...].T, in1_ref[...].T], axis=1)

    return pl.pallas_call(
        body,
        grid=(grid,),
        in_specs=[pl.BlockSpec((d, w), lambda i: (0, i)),
                  pl.BlockSpec((d, w),
                               lambda i: (0, jnp.minimum(i + grid, last)))],
        out_specs=pl.BlockSpec((w, 2 * d), lambda i: (i, 0)),
        out_shape=jax.ShapeDtypeStruct((h, 2 * d), jnp.float32),
    )(table_t, table_t), h


@functools.partial(jax.jit, static_argnums=(2, 3, 4, 5))
def _emb_lookup(xr, table, total, d, nch, half):
    mesh = plsc.VectorSubcoreMesh(core_axis_name="c", subcore_axis_name="s")
    b_per_w = nch * CH

    @functools.partial(
        pl.kernel,
        mesh=mesh,
        out_type=jax.ShapeDtypeStruct((total, d), jnp.float32),
        scratch_types=[
            pltpu.VMEM((nch, CH), jnp.int32),
            pltpu.VMEM((NB, CH, d), jnp.float32),
            pltpu.SemaphoreType.DMA,
            pltpu.SemaphoreType.DMA,
        ],
        compiler_params=pltpu.CompilerParams(use_tc_tiling_on_sc=False),
    )
    def k(x_hbm, tab_hbm, out_hbm, idx_v, rows_v, gsem, ssem):
        wid = lax.axis_index("s") * NC + lax.axis_index("c")
        base = wid * b_per_w
        pltpu.sync_copy(x_hbm.at[wid], idx_v)

        @pl.loop(0, nch)
        def _(c):
            row = idx_v.at[c]
            for g in range(CH // 16):
                vv = row[pl.ds(g * 16, 16)]
                hi = jnp.full((16,), 2 * half - 1, jnp.int32)
                lo = jnp.full((16,), 0, jnp.int32)
                off = jnp.where(vv >= jnp.full((16,), half, jnp.int32), hi, lo)
                row[pl.ds(g * 16, 16)] = vv + vv - off

        def gather(c, b):
            pltpu.async_copy(tab_hbm.at[idx_v.at[c]], rows_v.at[b], gsem)

        def wait_gather(b):
            pltpu.make_async_copy(
                tab_hbm.at[idx_v.at[0]], rows_v.at[b], gsem).wait()

        def wait_scatter():
            pltpu.make_async_copy(
                rows_v.at[0], out_hbm.at[pl.ds(base, CH)], ssem).wait()

        for b in range(NB):
            gather(b, b)

        @pl.loop(0, nch // NB)
        def _(p):
            for b in range(NB):
                s = p * NB + b
                wait_gather(b)
                pltpu.async_copy(
                    rows_v.at[b], out_hbm.at[pl.ds(base + s * CH, CH)], ssem)
                @pl.when(jnp.logical_and(s >= 2, s < nch - NB + 2))
                def _():
                    wait_scatter()
                    gather(s + NB - 2, (b - 2) % NB)

        for _ in range(NB):
            wait_scatter()

    return k(xr, table)


def kernel(x, embeddings):
    b, h = x.shape
    _, d = embeddings.shape
    total = b * h
    b_per_w = total // NW
    nch = b_per_w // CH
    xr = x.reshape(NW, nch, CH).astype(jnp.int32)
    v = embeddings.shape[0]
    table_lin, hh = _transpose_table_tc(embeddings.T, v, d)
    out = _emb_lookup(xr, table_lin.reshape(2 * hh, d), total, d, nch, hh)
    return out.reshape(b, h, d)


# speedup vs baseline: 4.1756x; 1.0268x over previous
import functools

import jax
import jax.numpy as jnp
from jax import lax
from jax.experimental import pallas as pl
from jax.experimental.pallas import tpu as pltpu
from jax.experimental.pallas import tpu_sc as plsc

NC = 2
NS = 16
NW = NC * NS

CH = 128
NB = 8


PAIR_W = 16384


def _transpose_table_tc(table_t, v, d):
    w = PAIR_W
    grid = (v + 2 * w - 1) // (2 * w)
    h = grid * w
    last = (v + w - 1) // w - 1

    def body(in0_ref, in1_ref, out_ref):
        out_ref[...] = jnp.concatenate(
            [in0_ref[...].T, in1_ref[...].T], axis=1)

    return pl.pallas_call(
        body,
        grid=(grid,),
        in_specs=[pl.BlockSpec((d, w), lambda i: (0, i)),
                  pl.BlockSpec((d, w),
                               lambda i: (0, jnp.minimum(i + grid, last)))],
        out_specs=pl.BlockSpec((w, 2 * d), lambda i: (i, 0)),
        out_shape=jax.ShapeDtypeStruct((h, 2 * d), jnp.float32),
    )(table_t, table_t), h


@functools.partial(jax.jit, static_argnums=(2, 3, 4, 5))
def _emb_lookup(xr, table, total, d, nch, half):
    mesh = plsc.VectorSubcoreMesh(core_axis_name="c", subcore_axis_name="s")
    b_per_w = nch * CH

    @functools.partial(
        pl.kernel,
        mesh=mesh,
        out_type=jax.ShapeDtypeStruct((total, d), jnp.float32),
        scratch_types=[
            pltpu.VMEM((nch, CH), jnp.int32),
            pltpu.VMEM((NB, CH, d), jnp.float32),
            pltpu.SemaphoreType.DMA,
            pltpu.SemaphoreType.DMA,
        ],
        compiler_params=pltpu.CompilerParams(use_tc_tiling_on_sc=False),
    )
    def k(x_hbm, tab_hbm, out_hbm, idx_v, rows_v, gsem, ssem):
        wid = lax.axis_index("s") * NC + lax.axis_index("c")
        base = wid * b_per_w
        pltpu.sync_copy(x_hbm.at[wid], idx_v)

        @pl.loop(0, nch)
        def _(c):
            row = idx_v.at[c]
            for g in range(CH // 16):
                vv = row[pl.ds(g * 16, 16)]
                hi = jnp.full((16,), 2 * half - 1, jnp.int32)
                lo = jnp.full((16,), 0, jnp.int32)
                off = jnp.where(vv >= jnp.full((16,), half, jnp.int32), hi, lo)
                row[pl.ds(g * 16, 16)] = vv + vv - off

        def gather(c, b):
            pltpu.async_copy(tab_hbm.at[idx_v.at[c]], rows_v.at[b], gsem)

        def wait_gather(b):
            pltpu.make_async_copy(
                tab_hbm.at[idx_v.at[0]], rows_v.at[b], gsem).wait()

        def wait_scatter():
            pltpu.make_async_copy(
                rows_v.at[0], out_hbm.at[pl.ds(base, CH)], ssem).wait()

        for b in range(NB):
            gather(b, b)

        @pl.loop(0, nch // NB)
        def _(p):
            for b in range(NB):
                s = p * NB + b
                wait_gather(b)
                pltpu.async_copy(
                    rows_v.at[b], out_hbm.at[pl.ds(base + s * CH, CH)], ssem)
                @pl.when(jnp.logical_and(s >= 2, s < nch - NB + 2))
                def _():
                    wait_scatter()
                    gather(s + NB - 2, (b - 2) % NB)

        for _ in range(NB):
            wait_scatter()

    return k(xr, table)


def kernel(x, embeddings):
    b, h = x.shape
    _, d = embeddings.shape
    total = b * h
    b_per_w = total // NW
    nch = b_per_w // CH
    xr = x.reshape(NW, nch, CH).astype(jnp.int32)
    v = embeddings.shape[0]
    table_lin, hh = _transpose_table_tc(embeddings.T, v, d)
    out = _emb_lookup(xr, table_lin.reshape(2 * hh, d), total, d, nch, hh)
    return out.reshape(b, h, d)
